# baseline (speedup 1.0000x reference)
import jax
import jax.numpy as jnp
from jax import lax
from jax.experimental import pallas as pl
from jax.experimental.pallas import tpu as pltpu
from jax.experimental.pallas import tpu_sc as plsc

_N = 10000
_E = 320000
_D = 128
_G = 64
_C = 10

_NP = 10240
_B = 128
_NCORES = 2
_NSUB = 16
_NCH = 80
_NCHH = 40
_NF = 160
_NS = 0
_FAST = 1
_ECH = _NCORES * _NSUB * _NCH
_EPAD = _ECH * _B
_RPT = _NP // _NSUB
_RB = 1024
_GRID = _NP // _RB

_mesh = plsc.VectorSubcoreMesh(core_axis_name="c", subcore_axis_name="s",
                               num_cores=_NCORES, num_subcores=_NSUB)


def _scatter_body(g_hbm, src_hbm, dst_hbm, zeros_hbm, out_hbm,
                  src_v, dst_v, b0, b1, acc_sh, sem0, sem1):
    c = lax.axis_index("c")
    s = lax.axis_index("s")
    r0 = s * _RPT
    pltpu.sync_copy(zeros_hbm.at[pl.ds(r0, _RPT)], acc_sh.at[pl.ds(r0, _RPT)])
    plsc.subcore_barrier()

    nch = jnp.where(c == _FAST, _NF, _NS)
    base = jnp.where(c == _FAST, s * _NF, _NSUB * _NF + s * _NS)
    for st in range(_NF // _NCHH):

        @pl.when(st * _NCHH < nch)
        def _():
            cb = base + st * _NCHH
            pltpu.sync_copy(src_hbm.at[pl.ds(cb, _NCHH)], src_v)
            pltpu.sync_copy(dst_hbm.at[pl.ds(cb, _NCHH)], dst_v)
            pltpu.async_copy(g_hbm.at[src_v.at[0]], b0, sem0)

            def body(t, carry):
                j = t * 2
                pltpu.async_copy(g_hbm.at[src_v.at[j + 1]], b1, sem1)
                pltpu.make_async_copy(g_hbm.at[src_v.at[j]], b0, sem0).wait()
                pltpu.sync_copy(b0, acc_sh.at[dst_v.at[j]], add=True)

                @pl.when(j + 2 < _NCHH)
                def _():
                    pltpu.async_copy(g_hbm.at[src_v.at[j + 2]], b0, sem0)

                pltpu.make_async_copy(g_hbm.at[src_v.at[j + 1]], b1, sem1).wait()
                pltpu.sync_copy(b1, acc_sh.at[dst_v.at[j + 1]], add=True)
                return carry

            lax.fori_loop(0, _NCHH // 2, body, 0)

    plsc.subcore_barrier()
    pltpu.sync_copy(acc_sh.at[pl.ds(r0, _RPT)], out_hbm.at[c, pl.ds(r0, _RPT)])


_sc_scatter = pl.kernel(
    _scatter_body,
    out_type=jax.ShapeDtypeStruct((_NCORES, _NP, _D), jnp.float32),
    mesh=_mesh,
    scratch_types=[
        pltpu.VMEM((_NCHH, _B), jnp.int32),
        pltpu.VMEM((_NCHH, _B), jnp.int32),
        pltpu.VMEM((_B, _D), jnp.float32),
        pltpu.VMEM((_B, _D), jnp.float32),
        pltpu.VMEM_SHARED((_NP, _D), jnp.float32),
        pltpu.SemaphoreType.DMA,
        pltpu.SemaphoreType.DMA,
    ],
)


def _hist_body(dst_hbm, ones_hbm, zeros_hbm, out_hbm, dst_v, ones_v, acc_sh, sem):
    c = lax.axis_index("c")
    s = lax.axis_index("s")
    r0 = s * _RPT
    pltpu.sync_copy(zeros_hbm.at[pl.ds(r0, _RPT)], acc_sh.at[pl.ds(r0, _RPT)])
    pltpu.sync_copy(ones_hbm, ones_v)
    cb = c * (_NSUB * _NCH) + s * _NCH
    pltpu.sync_copy(dst_hbm.at[pl.ds(cb, _NCH)], dst_v)
    plsc.subcore_barrier()

    def body(t, carry):
        j = t * 4
        d0 = pltpu.async_copy(ones_v, acc_sh.at[dst_v.at[j]], sem, add=True)
        d1 = pltpu.async_copy(ones_v, acc_sh.at[dst_v.at[j + 1]], sem, add=True)
        d2 = pltpu.async_copy(ones_v, acc_sh.at[dst_v.at[j + 2]], sem, add=True)
        d3 = pltpu.async_copy(ones_v, acc_sh.at[dst_v.at[j + 3]], sem, add=True)
        d0.wait(); d1.wait(); d2.wait(); d3.wait()
        return carry

    lax.fori_loop(0, _NCH // 4, body, 0)
    plsc.subcore_barrier()
    pltpu.sync_copy(acc_sh.at[pl.ds(r0, _RPT)], out_hbm.at[c, pl.ds(r0, _RPT)])


_sc_hist = pl.kernel(
    _hist_body,
    out_type=jax.ShapeDtypeStruct((_NCORES, _NP, _D), jnp.float32),
    mesh=_mesh,
    scratch_types=[
        pltpu.VMEM((_NCH, _B), jnp.int32),
        pltpu.VMEM((_B, _D), jnp.float32),
        pltpu.VMEM_SHARED((_NP, _D), jnp.float32),
        pltpu.SemaphoreType.DMA,
    ],
)


def _tc_first_body(x_ref, w_ref, cnt_ref, g_ref, dis_ref):
    deg = cnt_ref[0][:, 0:1] + cnt_ref[1][:, 0:1] + 1.0
    dis = lax.rsqrt(deg)
    g_ref[...] = dis * jnp.dot(x_ref[...], w_ref[...],
                               preferred_element_type=jnp.float32)
    dis_ref[...] = dis


_tc_first = pl.pallas_call(
    _tc_first_body,
    grid=(_GRID,),
    in_specs=[
        pl.BlockSpec((_RB, _D), lambda i: (i, 0)),
        pl.BlockSpec((_D, _D), lambda i: (0, 0)),
        pl.BlockSpec((_NCORES, _RB, _D), lambda i: (0, i, 0)),
    ],
    out_specs=[
        pl.BlockSpec((_RB, _D), lambda i: (i, 0)),
        pl.BlockSpec((_RB, 1), lambda i: (i, 0)),
    ],
    out_shape=[
        jax.ShapeDtypeStruct((_NP, _D), jnp.float32),
        jax.ShapeDtypeStruct((_NP, 1), jnp.float32),
    ],
)


def _tc_mid_body(acc_ref, g_ref, dis_ref, b_ref, w_ref, gn_ref):
    dis = dis_ref[...]
    t = acc_ref[0] + acc_ref[1] + g_ref[...]
    h = jnp.maximum(dis * t + b_ref[...], 0.0)
    row = pl.program_id(0) * _RB + lax.broadcasted_iota(jnp.int32, (_RB, 1), 0)
    h = jnp.where(row < _N, h, 0.0)
    gn_ref[...] = dis * jnp.dot(h, w_ref[...],
                                preferred_element_type=jnp.float32)


_tc_mid = pl.pallas_call(
    _tc_mid_body,
    grid=(_GRID,),
    in_specs=[
        pl.BlockSpec((_NCORES, _RB, _D), lambda i: (0, i, 0)),
        pl.BlockSpec((_RB, _D), lambda i: (i, 0)),
        pl.BlockSpec((_RB, 1), lambda i: (i, 0)),
        pl.BlockSpec((1, _D), lambda i: (0, 0)),
        pl.BlockSpec((_D, _D), lambda i: (0, 0)),
    ],
    out_specs=pl.BlockSpec((_RB, _D), lambda i: (i, 0)),
    out_shape=jax.ShapeDtypeStruct((_NP, _D), jnp.float32),
)


def _tc_final_body(acc_ref, g_ref, dis_ref, b_ref, batch_ref,
                   lw1_ref, lb1_ref, lw2_ref, lb2_ref, out_ref,
                   sums_ref, counts_ref):
    i = pl.program_id(0)

    @pl.when(i == 0)
    def _():
        sums_ref[...] = jnp.zeros_like(sums_ref)
        counts_ref[...] = jnp.zeros_like(counts_ref)

    dis = dis_ref[...]
    t = acc_ref[0] + acc_ref[1] + g_ref[...]
    h = jnp.maximum(dis * t + b_ref[...], 0.0)
    gid = lax.broadcasted_iota(jnp.int32, (_G, _RB), 0)
    onehot_t = (batch_ref[...] == gid).astype(jnp.float32)
    sums_ref[...] += jnp.dot(onehot_t, h, preferred_element_type=jnp.float32)
    counts_ref[...] += jnp.dot(onehot_t, jnp.ones((_RB, 1), jnp.float32),
                               preferred_element_type=jnp.float32)

    @pl.when(i == _GRID - 1)
    def _():
        pooled = sums_ref[...] / jnp.maximum(counts_ref[...], 1.0)
        z = jnp.maximum(jnp.dot(pooled, lw1_ref[...],
                                preferred_element_type=jnp.float32)
                        + lb1_ref[...], 0.0)
        z = jnp.dot(z, lw2_ref[...],
                    preferred_element_type=jnp.float32) + lb2_ref[...]
        m = jnp.max(z, axis=1, keepdims=True)
        e = jnp.exp(z - m)
        lse = jnp.log(jnp.sum(e, axis=1, keepdims=True)) + m
        out_ref[...] = z - lse


_tc_final = pl.pallas_call(
    _tc_final_body,
    grid=(_GRID,),
    in_specs=[
        pl.BlockSpec((_NCORES, _RB, _D), lambda i: (0, i, 0)),
        pl.BlockSpec((_RB, _D), lambda i: (i, 0)),
        pl.BlockSpec((_RB, 1), lambda i: (i, 0)),
        pl.BlockSpec((1, _D), lambda i: (0, 0)),
        pl.BlockSpec((1, _RB), lambda i: (0, i)),
        pl.BlockSpec((_D, _D), lambda i: (0, 0)),
        pl.BlockSpec((1, _D), lambda i: (0, 0)),
        pl.BlockSpec((_D, _C), lambda i: (0, 0)),
        pl.BlockSpec((1, _C), lambda i: (0, 0)),
    ],
    out_specs=pl.BlockSpec((_G, _C), lambda i: (0, 0)),
    out_shape=jax.ShapeDtypeStruct((_G, _C), jnp.float32),
    scratch_shapes=[
        pltpu.VMEM((_G, _D), jnp.float32),
        pltpu.VMEM((_G, 1), jnp.float32),
    ],
)


def kernel(x, edge_index, batch, W1, b1, W2, b2, W3, b3, LW1, Lb1, LW2, Lb2):
    pad = jnp.full((_EPAD - _E,), _N, jnp.int32)
    src2 = jnp.concatenate([edge_index[0], pad]).reshape(_ECH, _B)
    dst2 = jnp.concatenate([edge_index[1], pad]).reshape(_ECH, _B)
    x_pad = jnp.pad(x, ((0, _NP - _N), (0, 0)))
    batch_pad = jnp.concatenate(
        [batch.astype(jnp.int32), jnp.full((_NP - _N,), _G, jnp.int32)]
    ).reshape(1, _NP)
    zeros = jnp.zeros((_NP, _D), jnp.float32)
    ones = jnp.ones((_B, _D), jnp.float32)

    cnt = _sc_hist(dst2, ones, zeros)
    g1, dis = _tc_first(x_pad, W1, cnt)
    acc1 = _sc_scatter(g1, src2, dst2, zeros)
    g2 = _tc_mid(acc1, g1, dis, b1.reshape(1, _D), W2)
    acc2 = _sc_scatter(g2, src2, dst2, zeros)
    g3 = _tc_mid(acc2, g2, dis, b2.reshape(1, _D), W3)
    acc3 = _sc_scatter(g3, src2, dst2, zeros)
    return _tc_final(acc3, g3, dis, b3.reshape(1, _D), batch_pad,
                     LW1, Lb1.reshape(1, _D), LW2, Lb2.reshape(1, _C))


# speedup vs baseline: 1.1929x; 1.1929x over previous
import jax
import jax.numpy as jnp
from jax import lax
from jax.experimental import pallas as pl
from jax.experimental.pallas import tpu as pltpu
from jax.experimental.pallas import tpu_sc as plsc

_N = 10000
_E = 320000
_D = 128
_G = 64
_C = 10

_NP = 10240
_B = 128
_NCORES = 2
_NSUB = 16
_NCH = 80
_NCHH = 40
_NF = 120
_NS = 40
_FAST = 0
_ECH = _NCORES * _NSUB * _NCH
_EPAD = _ECH * _B
_RPT = _NP // _NSUB
_RB = 1024
_GRID = _NP // _RB

_mesh = plsc.VectorSubcoreMesh(core_axis_name="c", subcore_axis_name="s",
                               num_cores=_NCORES, num_subcores=_NSUB)


def _scatter_body(g_hbm, src_hbm, dst_hbm, zeros_hbm, out_hbm,
                  src_v, dst_v, b0, b1, acc_sh, sem0, sem1):
    c = lax.axis_index("c")
    s = lax.axis_index("s")
    r0 = s * _RPT
    pltpu.sync_copy(zeros_hbm.at[pl.ds(r0, _RPT)], acc_sh.at[pl.ds(r0, _RPT)])
    plsc.subcore_barrier()

    nch = jnp.where(c == _FAST, _NF, _NS)
    base = jnp.where(c == _FAST, s * _NF, _NSUB * _NF + s * _NS)
    for st in range(_NF // _NCHH):

        @pl.when(st * _NCHH < nch)
        def _():
            cb = base + st * _NCHH
            pltpu.sync_copy(src_hbm.at[pl.ds(cb, _NCHH)], src_v)
            pltpu.sync_copy(dst_hbm.at[pl.ds(cb, _NCHH)], dst_v)
            pltpu.async_copy(g_hbm.at[src_v.at[0]], b0, sem0)

            def body(t, carry):
                j = t * 2
                pltpu.async_copy(g_hbm.at[src_v.at[j + 1]], b1, sem1)
                pltpu.make_async_copy(g_hbm.at[src_v.at[j]], b0, sem0).wait()
                pltpu.sync_copy(b0, acc_sh.at[dst_v.at[j]], add=True)

                @pl.when(j + 2 < _NCHH)
                def _():
                    pltpu.async_copy(g_hbm.at[src_v.at[j + 2]], b0, sem0)

                pltpu.make_async_copy(g_hbm.at[src_v.at[j + 1]], b1, sem1).wait()
                pltpu.sync_copy(b1, acc_sh.at[dst_v.at[j + 1]], add=True)
                return carry

            lax.fori_loop(0, _NCHH // 2, body, 0)

    plsc.subcore_barrier()
    pltpu.sync_copy(acc_sh.at[pl.ds(r0, _RPT)], out_hbm.at[c, pl.ds(r0, _RPT)])


_sc_scatter = pl.kernel(
    _scatter_body,
    out_type=jax.ShapeDtypeStruct((_NCORES, _NP, _D), jnp.float32),
    mesh=_mesh,
    scratch_types=[
        pltpu.VMEM((_NCHH, _B), jnp.int32),
        pltpu.VMEM((_NCHH, _B), jnp.int32),
        pltpu.VMEM((_B, _D), jnp.float32),
        pltpu.VMEM((_B, _D), jnp.float32),
        pltpu.VMEM_SHARED((_NP, _D), jnp.float32),
        pltpu.SemaphoreType.DMA,
        pltpu.SemaphoreType.DMA,
    ],
)


def _hist_body(dst_hbm, ones_hbm, zeros_hbm, out_hbm, dst_v, ones_v, acc_sh, sem):
    c = lax.axis_index("c")
    s = lax.axis_index("s")
    r0 = s * _RPT
    pltpu.sync_copy(zeros_hbm.at[pl.ds(r0, _RPT)], acc_sh.at[pl.ds(r0, _RPT)])
    pltpu.sync_copy(ones_hbm, ones_v)
    cb = c * (_NSUB * _NCH) + s * _NCH
    pltpu.sync_copy(dst_hbm.at[pl.ds(cb, _NCH)], dst_v)
    plsc.subcore_barrier()

    def body(t, carry):
        j = t * 4
        d0 = pltpu.async_copy(ones_v, acc_sh.at[dst_v.at[j]], sem, add=True)
        d1 = pltpu.async_copy(ones_v, acc_sh.at[dst_v.at[j + 1]], sem, add=True)
        d2 = pltpu.async_copy(ones_v, acc_sh.at[dst_v.at[j + 2]], sem, add=True)
        d3 = pltpu.async_copy(ones_v, acc_sh.at[dst_v.at[j + 3]], sem, add=True)
        d0.wait(); d1.wait(); d2.wait(); d3.wait()
        return carry

    lax.fori_loop(0, _NCH // 4, body, 0)
    plsc.subcore_barrier()
    pltpu.sync_copy(acc_sh.at[pl.ds(r0, _RPT)], out_hbm.at[c, pl.ds(r0, _RPT)])


_sc_hist = pl.kernel(
    _hist_body,
    out_type=jax.ShapeDtypeStruct((_NCORES, _NP, _D), jnp.float32),
    mesh=_mesh,
    scratch_types=[
        pltpu.VMEM((_NCH, _B), jnp.int32),
        pltpu.VMEM((_B, _D), jnp.float32),
        pltpu.VMEM_SHARED((_NP, _D), jnp.float32),
        pltpu.SemaphoreType.DMA,
    ],
)


def _tc_first_body(x_ref, w_ref, cnt_ref, g_ref, dis_ref):
    deg = cnt_ref[0][:, 0:1] + cnt_ref[1][:, 0:1] + 1.0
    dis = lax.rsqrt(deg)
    g_ref[...] = dis * jnp.dot(x_ref[...], w_ref[...],
                               preferred_element_type=jnp.float32)
    dis_ref[...] = dis


_tc_first = pl.pallas_call(
    _tc_first_body,
    grid=(_GRID,),
    in_specs=[
        pl.BlockSpec((_RB, _D), lambda i: (i, 0)),
        pl.BlockSpec((_D, _D), lambda i: (0, 0)),
        pl.BlockSpec((_NCORES, _RB, _D), lambda i: (0, i, 0)),
    ],
    out_specs=[
        pl.BlockSpec((_RB, _D), lambda i: (i, 0)),
        pl.BlockSpec((_RB, 1), lambda i: (i, 0)),
    ],
    out_shape=[
        jax.ShapeDtypeStruct((_NP, _D), jnp.float32),
        jax.ShapeDtypeStruct((_NP, 1), jnp.float32),
    ],
)


def _tc_mid_body(acc_ref, g_ref, dis_ref, b_ref, w_ref, gn_ref):
    dis = dis_ref[...]
    t = acc_ref[0] + acc_ref[1] + g_ref[...]
    h = jnp.maximum(dis * t + b_ref[...], 0.0)
    row = pl.program_id(0) * _RB + lax.broadcasted_iota(jnp.int32, (_RB, 1), 0)
    h = jnp.where(row < _N, h, 0.0)
    gn_ref[...] = dis * jnp.dot(h, w_ref[...],
                                preferred_element_type=jnp.float32)


_tc_mid = pl.pallas_call(
    _tc_mid_body,
    grid=(_GRID,),
    in_specs=[
        pl.BlockSpec((_NCORES, _RB, _D), lambda i: (0, i, 0)),
        pl.BlockSpec((_RB, _D), lambda i: (i, 0)),
        pl.BlockSpec((_RB, 1), lambda i: (i, 0)),
        pl.BlockSpec((1, _D), lambda i: (0, 0)),
        pl.BlockSpec((_D, _D), lambda i: (0, 0)),
    ],
    out_specs=pl.BlockSpec((_RB, _D), lambda i: (i, 0)),
    out_shape=jax.ShapeDtypeStruct((_NP, _D), jnp.float32),
)


def _tc_final_body(acc_ref, g_ref, dis_ref, b_ref, batch_ref,
                   lw1_ref, lb1_ref, lw2_ref, lb2_ref, out_ref,
                   sums_ref, counts_ref):
    i = pl.program_id(0)

    @pl.when(i == 0)
    def _():
        sums_ref[...] = jnp.zeros_like(sums_ref)
        counts_ref[...] = jnp.zeros_like(counts_ref)

    dis = dis_ref[...]
    t = acc_ref[0] + acc_ref[1] + g_ref[...]
    h = jnp.maximum(dis * t + b_ref[...], 0.0)
    gid = lax.broadcasted_iota(jnp.int32, (_G, _RB), 0)
    onehot_t = (batch_ref[...] == gid).astype(jnp.float32)
    sums_ref[...] += jnp.dot(onehot_t, h, preferred_element_type=jnp.float32)
    counts_ref[...] += jnp.dot(onehot_t, jnp.ones((_RB, 1), jnp.float32),
                               preferred_element_type=jnp.float32)

    @pl.when(i == _GRID - 1)
    def _():
        pooled = sums_ref[...] / jnp.maximum(counts_ref[...], 1.0)
        z = jnp.maximum(jnp.dot(pooled, lw1_ref[...],
                                preferred_element_type=jnp.float32)
                        + lb1_ref[...], 0.0)
        z = jnp.dot(z, lw2_ref[...],
                    preferred_element_type=jnp.float32) + lb2_ref[...]
        m = jnp.max(z, axis=1, keepdims=True)
        e = jnp.exp(z - m)
        lse = jnp.log(jnp.sum(e, axis=1, keepdims=True)) + m
        out_ref[...] = z - lse


_tc_final = pl.pallas_call(
    _tc_final_body,
    grid=(_GRID,),
    in_specs=[
        pl.BlockSpec((_NCORES, _RB, _D), lambda i: (0, i, 0)),
        pl.BlockSpec((_RB, _D), lambda i: (i, 0)),
        pl.BlockSpec((_RB, 1), lambda i: (i, 0)),
        pl.BlockSpec((1, _D), lambda i: (0, 0)),
        pl.BlockSpec((1, _RB), lambda i: (0, i)),
        pl.BlockSpec((_D, _D), lambda i: (0, 0)),
        pl.BlockSpec((1, _D), lambda i: (0, 0)),
        pl.BlockSpec((_D, _C), lambda i: (0, 0)),
        pl.BlockSpec((1, _C), lambda i: (0, 0)),
    ],
    out_specs=pl.BlockSpec((_G, _C), lambda i: (0, 0)),
    out_shape=jax.ShapeDtypeStruct((_G, _C), jnp.float32),
    scratch_shapes=[
        pltpu.VMEM((_G, _D), jnp.float32),
        pltpu.VMEM((_G, 1), jnp.float32),
    ],
)


def kernel(x, edge_index, batch, W1, b1, W2, b2, W3, b3, LW1, Lb1, LW2, Lb2):
    pad = jnp.full((_EPAD - _E,), _N, jnp.int32)
    src2 = jnp.concatenate([edge_index[0], pad]).reshape(_ECH, _B)
    dst2 = jnp.concatenate([edge_index[1], pad]).reshape(_ECH, _B)
    x_pad = jnp.pad(x, ((0, _NP - _N), (0, 0)))
    batch_pad = jnp.concatenate(
        [batch.astype(jnp.int32), jnp.full((_NP - _N,), _G, jnp.int32)]
    ).reshape(1, _NP)
    zeros = jnp.zeros((_NP, _D), jnp.float32)
    ones = jnp.ones((_B, _D), jnp.float32)

    cnt = _sc_hist(dst2, ones, zeros)
    g1, dis = _tc_first(x_pad, W1, cnt)
    acc1 = _sc_scatter(g1, src2, dst2, zeros)
    g2 = _tc_mid(acc1, g1, dis, b1.reshape(1, _D), W2)
    acc2 = _sc_scatter(g2, src2, dst2, zeros)
    g3 = _tc_mid(acc2, g2, dis, b2.reshape(1, _D), W3)
    acc3 = _sc_scatter(g3, src2, dst2, zeros)
    return _tc_final(acc3, g3, dis, b3.reshape(1, _D), batch_pad,
                     LW1, Lb1.reshape(1, _D), LW2, Lb2.reshape(1, _C))
